# baseline (speedup 1.0000x reference)
import functools

import jax
import jax.numpy as jnp
from jax import lax
from jax.experimental import pallas as pl
from jax.experimental.pallas import tpu as pltpu
from jax.experimental.pallas import tpu_sc as plsc

_BATCH = 4096
_HIST = 200
_D = 64
_B = _BATCH * _HIST
_NC = 2
_NS = 16
_NW = _NC * _NS
_BPW = _B // _NW
_CHUNK = 128
_NCHUNK = _BPW // _CHUNK


def _make_gather():
  mesh = plsc.VectorSubcoreMesh(core_axis_name="c", subcore_axis_name="s")

  @functools.partial(
      pl.kernel,
      mesh=mesh,
      out_type=jax.ShapeDtypeStruct((_B, _D), jnp.float32),
      compiler_params=pltpu.CompilerParams(use_tc_tiling_on_sc=False),
      scratch_types=[
          pltpu.VMEM((_NCHUNK, _CHUNK), jnp.int32),
          pltpu.VMEM((_CHUNK, _D), jnp.float32),
          pltpu.SemaphoreType.DMA,
      ],
  )
  def gather_kernel(x_hbm, table_hbm, out_hbm, idx_v, rows_v, sem):
    wid = lax.axis_index("s") * _NC + lax.axis_index("c")
    base = wid * _BPW
    pltpu.sync_copy(x_hbm.at[wid], idx_v)

    def body(i, carry):
      pltpu.async_copy(table_hbm.at[idx_v.at[i]], rows_v, sem).wait()
      pltpu.sync_copy(rows_v, out_hbm.at[pl.ds(base + i * _CHUNK, _CHUNK)])
      return carry

    lax.fori_loop(0, _NCHUNK, body, 0)

  return gather_kernel


_gather = _make_gather()


def kernel(x, table):
  xw = x.reshape(_NW, _NCHUNK, _CHUNK)
  out = _gather(xw, table)
  return out.reshape(_BATCH, _HIST, _D)


# speedup vs baseline: 1.1097x; 1.1097x over previous
import functools

import jax
import jax.numpy as jnp
from jax import lax
from jax.experimental import pallas as pl
from jax.experimental.pallas import tpu as pltpu
from jax.experimental.pallas import tpu_sc as plsc

_BATCH = 4096
_HIST = 200
_D = 64
_B = _BATCH * _HIST
_NC = 2
_NS = 16
_NW = _NC * _NS
_BPW = _B // _NW
_CHUNK = 128
_NCHUNK = _BPW // _CHUNK
_NBUF = 8
_LEAD = _NBUF // 2
_NOUTER = _NCHUNK // _NBUF


def _make_gather():
  mesh = plsc.VectorSubcoreMesh(core_axis_name="c", subcore_axis_name="s")

  @functools.partial(
      pl.kernel,
      mesh=mesh,
      out_type=jax.ShapeDtypeStruct((_B, _D), jnp.float32),
      compiler_params=pltpu.CompilerParams(use_tc_tiling_on_sc=False),
      scratch_types=[
          pltpu.VMEM((_NCHUNK, _CHUNK), jnp.int32),
          pltpu.VMEM((_NBUF, _CHUNK, _D), jnp.float32),
          pltpu.SemaphoreType.DMA((_NBUF,)),
          pltpu.SemaphoreType.DMA((_NBUF,)),
      ],
  )
  def gather_kernel(x_hbm, table_hbm, out_hbm, idx_v, rows_v, gsem, wsem):
    wid = lax.axis_index("s") * _NC + lax.axis_index("c")
    base = wid * _BPW
    pltpu.sync_copy(x_hbm.at[wid], idx_v)

    def start_gather(t, b):
      pltpu.async_copy(table_hbm.at[idx_v.at[t]], rows_v.at[b], gsem.at[b])

    def wait_gather(b):
      pltpu.make_async_copy(
          table_hbm.at[idx_v.at[0]], rows_v.at[b], gsem.at[b]).wait()

    def start_write(t, b):
      pltpu.async_copy(
          rows_v.at[b], out_hbm.at[pl.ds(base + t * _CHUNK, _CHUNK)],
          wsem.at[b])

    def wait_write(b):
      pltpu.make_async_copy(
          rows_v.at[b], out_hbm.at[pl.ds(base, _CHUNK)], wsem.at[b]).wait()

    for b in range(_LEAD):
      start_gather(b, b)

    def body(g, carry):
      for b in range(_NBUF):
        s = g * _NBUF + b
        bg = (b + _LEAD) % _NBUF
        if b >= _LEAD:
          wait_write(bg)
        else:
          @pl.when(g > 0)
          def _():
            wait_write(bg)
        if b < _NBUF - _LEAD:
          start_gather(s + _LEAD, bg)
        else:
          @pl.when(g < _NOUTER - 1)
          def _():
            start_gather(s + _LEAD, bg)
        wait_gather(b)
        start_write(s, b)
      return carry

    lax.fori_loop(0, _NOUTER, body, 0)

    for b in range(_NBUF - _LEAD, _NBUF):
      wait_write(b)

  return gather_kernel


_gather = _make_gather()


def kernel(x, table):
  xw = x.reshape(_NW, _NCHUNK, _CHUNK)
  out = _gather(xw, table)
  return out.reshape(_BATCH, _HIST, _D)


# speedup vs baseline: 1.3707x; 1.2352x over previous
import functools

import jax
import jax.numpy as jnp
from jax import lax
from jax.experimental import pallas as pl
from jax.experimental.pallas import tpu as pltpu
from jax.experimental.pallas import tpu_sc as plsc

_BATCH = 4096
_HIST = 200
_D = 64
_DP = 128
_V = 1000000
_B = _BATCH * _HIST
_NC = 2
_NS = 16
_NW = _NC * _NS
_BPW = _B // _NW
_CHUNK = 128
_NCHUNK = _BPW // _CHUNK
_NBUF = 4
_LEAD = _NBUF // 2
_NOUTER = _NCHUNK // _NBUF
_TBLK = 2048


def _relayout_body(tn_ref, out_ref):
  t = jnp.swapaxes(tn_ref[...], 0, 1)
  out_ref[...] = jnp.concatenate([t, t], axis=1)


_relayout = pl.pallas_call(
    _relayout_body,
    grid=(pl.cdiv(_V, _TBLK),),
    in_specs=[pl.BlockSpec((_D, _TBLK), lambda i: (0, i))],
    out_specs=pl.BlockSpec((_TBLK, _DP), lambda i: (i, 0)),
    out_shape=jax.ShapeDtypeStruct((_V, _DP), jnp.float32),
)


def _make_gather():
  mesh = plsc.VectorSubcoreMesh(core_axis_name="c", subcore_axis_name="s")

  @functools.partial(
      pl.kernel,
      mesh=mesh,
      out_type=jax.ShapeDtypeStruct((_B, _DP), jnp.float32),
      scratch_types=[
          pltpu.VMEM((_NCHUNK, _CHUNK), jnp.int32),
          pltpu.VMEM((_NBUF, _CHUNK, _DP), jnp.float32),
          pltpu.SemaphoreType.DMA((_NBUF,)),
          pltpu.SemaphoreType.DMA((_NBUF,)),
      ],
  )
  def gather_kernel(x_hbm, table_hbm, out_hbm, idx_v, rows_v, gsem, wsem):
    wid = lax.axis_index("s") * _NC + lax.axis_index("c")
    base = wid * _BPW
    pltpu.sync_copy(x_hbm.at[wid], idx_v)

    def start_gather(t, b):
      pltpu.async_copy(table_hbm.at[idx_v.at[t]], rows_v.at[b], gsem.at[b])

    def wait_gather(b):
      pltpu.make_async_copy(
          table_hbm.at[idx_v.at[0]], rows_v.at[b], gsem.at[b]).wait()

    def start_write(t, b):
      pltpu.async_copy(
          rows_v.at[b], out_hbm.at[pl.ds(base + t * _CHUNK, _CHUNK)],
          wsem.at[b])

    def wait_write(b):
      pltpu.make_async_copy(
          rows_v.at[b], out_hbm.at[pl.ds(base, _CHUNK)], wsem.at[b]).wait()

    for b in range(_LEAD):
      start_gather(b, b)

    def body(g, carry):
      for b in range(_NBUF):
        s = g * _NBUF + b
        bg = (b + _LEAD) % _NBUF
        if b >= _LEAD:
          wait_write(bg)
        else:
          @pl.when(g > 0)
          def _():
            wait_write(bg)
        if b < _NBUF - _LEAD:
          start_gather(s + _LEAD, bg)
        else:
          @pl.when(g < _NOUTER - 1)
          def _():
            start_gather(s + _LEAD, bg)
        wait_gather(b)
        start_write(s, b)
      return carry

    lax.fori_loop(0, _NOUTER, body, 0)

    for b in range(_NBUF - _LEAD, _NBUF):
      wait_write(b)

  return gather_kernel


_gather = _make_gather()


def kernel(x, table):
  tp = _relayout(table.T)
  xw = x.reshape(_NW, _NCHUNK, _CHUNK)
  out = _gather(xw, tp)
  return out[:, :_D].reshape(_BATCH, _HIST, _D)


# speedup vs baseline: 1.6686x; 1.2174x over previous
import functools

import jax
import jax.numpy as jnp
from jax import lax
from jax.experimental import pallas as pl
from jax.experimental.pallas import tpu as pltpu
from jax.experimental.pallas import tpu_sc as plsc

_BATCH = 4096
_HIST = 200
_D = 64
_DP = 128
_V = 1000000
_B = _BATCH * _HIST
_NC = 2
_NS = 16
_NW = _NC * _NS
_BPW = _B // _NW
_CHUNK = 128
_NCHUNK = _BPW // _CHUNK
_NBUF = 4
_LEAD = _NBUF // 2
_NOUTER = _NCHUNK // _NBUF
_TBLK = 4096


def _relayout_body(tn_ref, out_ref):
  out_ref[:, :_D] = jnp.swapaxes(tn_ref[...], 0, 1)


_relayout = pl.pallas_call(
    _relayout_body,
    grid=(pl.cdiv(_V, _TBLK),),
    in_specs=[pl.BlockSpec((_D, _TBLK), lambda i: (0, i))],
    out_specs=pl.BlockSpec((_TBLK, _DP), lambda i: (i, 0)),
    out_shape=jax.ShapeDtypeStruct((_V, _DP), jnp.float32),
)


def _make_gather():
  mesh = plsc.VectorSubcoreMesh(core_axis_name="c", subcore_axis_name="s")

  @functools.partial(
      pl.kernel,
      mesh=mesh,
      out_type=jax.ShapeDtypeStruct((_B, _DP), jnp.float32),
      scratch_types=[
          pltpu.VMEM((_NCHUNK, _CHUNK), jnp.int32),
          pltpu.VMEM((_NBUF, _CHUNK, _DP), jnp.float32),
          pltpu.SemaphoreType.DMA((_NBUF,)),
          pltpu.SemaphoreType.DMA((_NBUF,)),
      ],
  )
  def gather_kernel(x_hbm, table_hbm, out_hbm, idx_v, rows_v, gsem, wsem):
    wid = lax.axis_index("s") * _NC + lax.axis_index("c")
    base = wid * _BPW
    pltpu.sync_copy(x_hbm.at[wid], idx_v)

    def start_gather(t, b):
      pltpu.async_copy(table_hbm.at[idx_v.at[t]], rows_v.at[b], gsem.at[b])

    def wait_gather(b):
      pltpu.make_async_copy(
          table_hbm.at[idx_v.at[0]], rows_v.at[b], gsem.at[b]).wait()

    def start_write(t, b):
      pltpu.async_copy(
          rows_v.at[b], out_hbm.at[pl.ds(base + t * _CHUNK, _CHUNK)],
          wsem.at[b])

    def wait_write(b):
      pltpu.make_async_copy(
          rows_v.at[b], out_hbm.at[pl.ds(base, _CHUNK)], wsem.at[b]).wait()

    for b in range(_LEAD):
      start_gather(b, b)

    def body(g, carry):
      for b in range(_NBUF):
        s = g * _NBUF + b
        bg = (b + _LEAD) % _NBUF
        if b >= _LEAD:
          wait_write(bg)
        else:
          @pl.when(g > 0)
          def _():
            wait_write(bg)
        if b < _NBUF - _LEAD:
          start_gather(s + _LEAD, bg)
        else:
          @pl.when(g < _NOUTER - 1)
          def _():
            start_gather(s + _LEAD, bg)
        wait_gather(b)
        start_write(s, b)
      return carry

    lax.fori_loop(0, _NOUTER, body, 0)

    for b in range(_NBUF - _LEAD, _NBUF):
      wait_write(b)

  return gather_kernel


_gather = _make_gather()


def kernel(x, table):
  tp = _relayout(table.T)
  xw = x.reshape(_NW, _NCHUNK, _CHUNK)
  out = _gather(xw, tp)
  return out[:, :_D].reshape(_BATCH, _HIST, _D)


# speedup vs baseline: 1.8233x; 1.0927x over previous
import functools

import jax
import jax.numpy as jnp
from jax import lax
from jax.experimental import pallas as pl
from jax.experimental.pallas import tpu as pltpu
from jax.experimental.pallas import tpu_sc as plsc

_BATCH = 4096
_HIST = 200
_D = 64
_DP = 128
_V = 1000000
_B = _BATCH * _HIST
_NC = 2
_NS = 16
_NW = _NC * _NS
_BPW = _B // _NW
_CHUNK = 128
_NCHUNK = _BPW // _CHUNK
_NBUF = 4
_LEAD = _NBUF // 2
_NOUTER = _NCHUNK // _NBUF
_TBLK = 8192


def _relayout_body(tn_ref, out_ref):
  out_ref[:, :_D] = jnp.swapaxes(tn_ref[...], 0, 1)


_relayout = pl.pallas_call(
    _relayout_body,
    grid=(pl.cdiv(_V, _TBLK),),
    in_specs=[pl.BlockSpec((_D, _TBLK), lambda i: (0, i))],
    out_specs=pl.BlockSpec((_TBLK, _DP), lambda i: (i, 0)),
    out_shape=jax.ShapeDtypeStruct((_V, _DP), jnp.float32),
)


def _make_gather():
  mesh = plsc.VectorSubcoreMesh(core_axis_name="c", subcore_axis_name="s")

  @functools.partial(
      pl.kernel,
      mesh=mesh,
      out_type=jax.ShapeDtypeStruct((_B, _DP), jnp.float32),
      scratch_types=[
          pltpu.VMEM((_NCHUNK, _CHUNK), jnp.int32),
          pltpu.VMEM((_NBUF, _CHUNK, _DP), jnp.float32),
          pltpu.SemaphoreType.DMA((_NBUF,)),
          pltpu.SemaphoreType.DMA((_NBUF,)),
      ],
  )
  def gather_kernel(x_hbm, table_hbm, out_hbm, idx_v, rows_v, gsem, wsem):
    wid = lax.axis_index("s") * _NC + lax.axis_index("c")
    base = wid * _BPW
    pltpu.sync_copy(x_hbm.at[wid], idx_v)

    def start_gather(t, b):
      pltpu.async_copy(table_hbm.at[idx_v.at[t]], rows_v.at[b], gsem.at[b])

    def wait_gather(b):
      pltpu.make_async_copy(
          table_hbm.at[idx_v.at[0]], rows_v.at[b], gsem.at[b]).wait()

    def start_write(t, b):
      pltpu.async_copy(
          rows_v.at[b], out_hbm.at[pl.ds(base + t * _CHUNK, _CHUNK)],
          wsem.at[b])

    def wait_write(b):
      pltpu.make_async_copy(
          rows_v.at[b], out_hbm.at[pl.ds(base, _CHUNK)], wsem.at[b]).wait()

    for b in range(_LEAD):
      start_gather(b, b)

    def body(g, carry):
      for b in range(_NBUF):
        s = g * _NBUF + b
        bg = (b + _LEAD) % _NBUF
        if b >= _LEAD:
          wait_write(bg)
        else:
          @pl.when(g > 0)
          def _():
            wait_write(bg)
        if b < _NBUF - _LEAD:
          start_gather(s + _LEAD, bg)
        else:
          @pl.when(g < _NOUTER - 1)
          def _():
            start_gather(s + _LEAD, bg)
        wait_gather(b)
        start_write(s, b)
      return carry

    lax.fori_loop(0, _NOUTER, body, 0)

    for b in range(_NBUF - _LEAD, _NBUF):
      wait_write(b)

  return gather_kernel


_gather = _make_gather()


def kernel(x, table):
  tp = _relayout(table.T)
  xw = x.reshape(_NW, _NCHUNK, _CHUNK)
  out = _gather(xw, tp)
  return out[:, :_D].reshape(_BATCH, _HIST, _D)


# speedup vs baseline: 1.8246x; 1.0007x over previous
import functools

import jax
import jax.numpy as jnp
from jax import lax
from jax.experimental import pallas as pl
from jax.experimental.pallas import tpu as pltpu
from jax.experimental.pallas import tpu_sc as plsc

_BATCH = 4096
_HIST = 200
_D = 64
_DP = 128
_V = 1000000
_B = _BATCH * _HIST
_NC = 2
_NS = 16
_NW = _NC * _NS
_BPW = _B // _NW
_CHUNK = 128
_NCHUNK = _BPW // _CHUNK
_NBUF = 5
_LEAD = 2
_NOUTER = _NCHUNK // _NBUF
_TBLK = 8192


def _relayout_body(tn_ref, out_ref):
  out_ref[:, :_D] = jnp.swapaxes(tn_ref[...], 0, 1)


_relayout = pl.pallas_call(
    _relayout_body,
    grid=(pl.cdiv(_V, _TBLK),),
    in_specs=[pl.BlockSpec((_D, _TBLK), lambda i: (0, i))],
    out_specs=pl.BlockSpec((_TBLK, _DP), lambda i: (i, 0)),
    out_shape=jax.ShapeDtypeStruct((_V, _DP), jnp.float32),
)


def _make_gather():
  mesh = plsc.VectorSubcoreMesh(core_axis_name="c", subcore_axis_name="s")

  @functools.partial(
      pl.kernel,
      mesh=mesh,
      out_type=jax.ShapeDtypeStruct((_B, _DP), jnp.float32),
      scratch_types=[
          pltpu.VMEM((_NCHUNK, _CHUNK), jnp.int32),
          pltpu.VMEM((_NBUF, _CHUNK, _DP), jnp.float32),
          pltpu.SemaphoreType.DMA((_NBUF,)),
          pltpu.SemaphoreType.DMA((_NBUF,)),
      ],
  )
  def gather_kernel(x_hbm, table_hbm, out_hbm, idx_v, rows_v, gsem, wsem):
    wid = lax.axis_index("s") * _NC + lax.axis_index("c")
    base = wid * _BPW
    pltpu.sync_copy(x_hbm.at[wid], idx_v)

    def start_gather(t, b):
      pltpu.async_copy(table_hbm.at[idx_v.at[t]], rows_v.at[b], gsem.at[b])

    def wait_gather(b):
      pltpu.make_async_copy(
          table_hbm.at[idx_v.at[0]], rows_v.at[b], gsem.at[b]).wait()

    def start_write(t, b):
      pltpu.async_copy(
          rows_v.at[b], out_hbm.at[pl.ds(base + t * _CHUNK, _CHUNK)],
          wsem.at[b])

    def wait_write(b):
      pltpu.make_async_copy(
          rows_v.at[b], out_hbm.at[pl.ds(base, _CHUNK)], wsem.at[b]).wait()

    for b in range(_LEAD):
      start_gather(b, b)

    def body(g, carry):
      for b in range(_NBUF):
        s = g * _NBUF + b
        bg = (b + _LEAD) % _NBUF
        if b >= _NBUF - _LEAD:
          wait_write(bg)
        else:
          @pl.when(g > 0)
          def _():
            wait_write(bg)
        if b < _NBUF - _LEAD:
          start_gather(s + _LEAD, bg)
        else:
          @pl.when(g < _NOUTER - 1)
          def _():
            start_gather(s + _LEAD, bg)
        wait_gather(b)
        start_write(s, b)
      return carry

    lax.fori_loop(0, _NOUTER, body, 0)

    for t in range(_NCHUNK + _LEAD - _NBUF, _NCHUNK):
      wait_write(t % _NBUF)

  return gather_kernel


_gather = _make_gather()


def kernel(x, table):
  tp = _relayout(table.T)
  xw = x.reshape(_NW, _NCHUNK, _CHUNK)
  out = _gather(xw, tp)
  return out[:, :_D].reshape(_BATCH, _HIST, _D)
